# baseline (speedup 1.0000x reference)
import jax
import jax.numpy as jnp
from jax.experimental import pallas as pl
from jax.experimental.pallas import tpu as pltpu

_NBUF = 4


def _pick_block(n: int) -> int:
    for r in range(min(n, 256), 7, -1):
        if n % r == 0 and r % 8 == 0:
            return r
    return n


def _make_body(n, r, nchunks):
    t = 2 * nchunks

    def _body(adj_ref, x_ref, w1_ref, b1_ref, w2_ref, b2_ref, o_ref,
              x_sc, xw_sc, hw_sc, bufs, sems, x_sem):
        def start(c, slot):
            layer = c // nchunks
            i = c - layer * nchunks
            pltpu.make_async_copy(
                adj_ref.at[layer, pl.ds(i * r, r), :],
                bufs.at[slot],
                sems.at[slot],
            ).start()

        pltpu.make_async_copy(x_ref, x_sc, x_sem).start()
        for w in range(min(_NBUF, t)):
            start(w, w)

        pltpu.make_async_copy(x_ref, x_sc, x_sem).wait()
        xw_sc[...] = jnp.dot(x_sc[...], w1_ref[...],
                             preferred_element_type=jnp.float32)

        def step(c, carry):
            slot = jax.lax.rem(c, _NBUF)
            layer = c // nchunks
            i = c - layer * nchunks
            pltpu.make_async_copy(
                adj_ref.at[layer, pl.ds(i * r, r), :],
                bufs.at[slot],
                sems.at[slot],
            ).wait()

            @pl.when(layer == 0)
            def _layer1():
                h = jnp.dot(bufs[slot], xw_sc[...],
                            preferred_element_type=jnp.float32)
                h = jnp.maximum(h + b1_ref[...], 0.0)
                hw_sc[pl.ds(i * r, r), :] = jnp.dot(
                    h, w2_ref[...], preferred_element_type=jnp.float32)

            @pl.when(layer == 1)
            def _layer2():
                logits = jnp.dot(bufs[slot], hw_sc[...],
                                 preferred_element_type=jnp.float32)
                logits = logits + b2_ref[...]
                m = jnp.max(logits, axis=-1, keepdims=True)
                s = logits - m
                lse = jnp.log(jnp.sum(jnp.exp(s), axis=-1, keepdims=True))
                o_ref[pl.ds(i * r, r), :] = s - lse

            nc = c + _NBUF

            @pl.when(nc < t)
            def _prefetch():
                nl = nc // nchunks
                ni = nc - nl * nchunks
                pltpu.make_async_copy(
                    adj_ref.at[nl, pl.ds(ni * r, r), :],
                    bufs.at[slot],
                    sems.at[slot],
                ).start()

            return carry

        jax.lax.fori_loop(0, t, step, 0)

    return _body


def kernel(x, adjs, W1, b1, W2, b2):
    n, in_feats = x.shape
    h_feats = W1.shape[1]
    num_classes = W2.shape[1]
    r = _pick_block(n)
    nchunks = n // r
    b1r = b1.reshape(1, h_feats)
    b2r = b2.reshape(1, num_classes)

    return pl.pallas_call(
        _make_body(n, r, nchunks),
        in_specs=[
            pl.BlockSpec(memory_space=pltpu.MemorySpace.HBM),
            pl.BlockSpec(memory_space=pltpu.MemorySpace.HBM),
            pl.BlockSpec(memory_space=pltpu.MemorySpace.VMEM),
            pl.BlockSpec(memory_space=pltpu.MemorySpace.VMEM),
            pl.BlockSpec(memory_space=pltpu.MemorySpace.VMEM),
            pl.BlockSpec(memory_space=pltpu.MemorySpace.VMEM),
        ],
        out_specs=pl.BlockSpec(memory_space=pltpu.MemorySpace.VMEM),
        out_shape=jax.ShapeDtypeStruct((n, num_classes), jnp.float32),
        scratch_shapes=[
            pltpu.VMEM((n, in_feats), jnp.float32),
            pltpu.VMEM((n, h_feats), jnp.float32),
            pltpu.VMEM((n, num_classes), jnp.float32),
            pltpu.VMEM((_NBUF, r, n), jnp.float32),
            pltpu.SemaphoreType.DMA((_NBUF,)),
            pltpu.SemaphoreType.DMA,
        ],
        compiler_params=pltpu.CompilerParams(
            vmem_limit_bytes=100 * 1024 * 1024,
        ),
    )(adjs, x, W1, b1r, W2, b2r)
